# baseline (speedup 1.0000x reference)
import jax
import jax.numpy as jnp
from jax import lax
from jax.experimental import pallas as pl
from jax.experimental.pallas import tpu as pltpu
from jax.experimental.pallas import tpu_sc as plsc

B, L, D = 16, 2048, 1024
LANES = 16
HALF = D // 2
NVEC = HALF // LANES
CH = 64
SEQ_PER_SC = B // 2
WPH = 8


def _body(xs_hbm, len_hbm, out_hbm, part_hbm, lbuf, buf0, buf1,
          obuf, sem0, sem1, semw):
    c = lax.axis_index("c")
    s = lax.axis_index("s")
    h = s % 2
    wk = s // 2
    col0 = h * HALF

    pltpu.sync_copy(len_hbm, lbuf)
    lvec = lbuf[...]
    lall = [lvec[i] for i in range(B)]
    ls = [jnp.where(c == 0, lall[j], lall[SEQ_PER_SC + j])
          for j in range(SEQ_PER_SC)]
    cum = [jnp.int32(0)]
    for j in range(SEQ_PER_SC):
        cum.append(cum[j] + ls[j])
    tot = cum[SEQ_PER_SC]
    lo = wk * tot // WPH
    hi = (wk + 1) * tot // WPH

    def issue(t0, buf, sem):
        t0c = jnp.minimum(t0, L - CH)
        pltpu.make_async_copy(
            xs_hbm.at[b, pl.ds(t0c, CH), pl.ds(col0, HALF)], buf, sem
        ).start()

    def wait(buf, sem):
        pltpu.make_async_copy(
            xs_hbm.at[b, pl.ds(0, CH), pl.ds(col0, HALF)], buf, sem
        ).wait()

    for j in range(SEQ_PER_SC):
        b = c * SEQ_PER_SC + j
        t_lo = jnp.clip(lo - cum[j], 0, ls[j])
        t_end = jnp.clip(hi - cum[j], 0, ls[j])
        t_len = t_end - t_lo
        al_lo = (t_lo // 8) * 8
        nch = jnp.where(t_len > 0, (t_end - al_lo + CH - 1) // CH, 0)

        def accum(i, buf, accs):
            t0 = al_lo + i * CH
            t0c = jnp.minimum(t0, L - CH)
            start = jnp.maximum(t_lo, t0)
            off = start - t0c
            nv = jnp.clip(jnp.minimum(t_end, t0 + CH) - start, 0, CH)

            def pair_rows(r2, a):
                r0 = 2 * r2
                keep = r0 + 1 < nv
                row0 = off + r0
                row1 = jnp.minimum(row0 + 1, CH - 1)
                a = tuple(
                    a[v] + buf[row0, pl.ds(v * LANES, LANES)]
                    for v in range(NVEC)
                )
                zero = jnp.zeros((LANES,), jnp.float32)
                return tuple(
                    a[v] + jnp.where(keep, buf[row1, pl.ds(v * LANES, LANES)], zero)
                    for v in range(NVEC)
                )

            return lax.fori_loop(0, (nv + 1) // 2, pair_rows, accs)

        @pl.when(0 < nch)
        def _():
            issue(al_lo, buf0, sem0)

        @pl.when(1 < nch)
        def _():
            issue(al_lo + CH, buf1, sem1)

        def pair_body(i2, accs):
            ca = 2 * i2
            wait(buf0, sem0)
            accs = accum(ca, buf0, accs)

            @pl.when(ca + 2 < nch)
            def _():
                issue(al_lo + (ca + 2) * CH, buf0, sem0)

            @pl.when(ca + 1 < nch)
            def _():
                wait(buf1, sem1)

            accs = accum(ca + 1, buf1, accs)

            @pl.when(ca + 3 < nch)
            def _():
                issue(al_lo + (ca + 3) * CH, buf1, sem1)

            return accs

        accs = tuple(jnp.zeros((LANES,), jnp.float32) for _ in range(NVEC))
        accs = lax.fori_loop(0, (nch + 1) // 2, pair_body, accs)
        for v in range(NVEC):
            obuf[j, pl.ds(v * LANES, LANES)] = accs[v]
        pltpu.make_async_copy(obuf.at[j], part_hbm.at[b, h, wk], semw).start()

    for j in range(SEQ_PER_SC):
        pltpu.make_async_copy(
            obuf.at[j], part_hbm.at[c * SEQ_PER_SC + j, h, wk], semw
        ).wait()

    plsc.subcore_barrier()

    b_f = c * SEQ_PER_SC + wk
    l_f = ls[0]
    for j in range(1, SEQ_PER_SC):
        l_f = jnp.where(wk == j, ls[j], l_f)
    inv = 1.0 / jnp.full((LANES,), l_f).astype(jnp.float32)
    pltpu.sync_copy(part_hbm.at[b_f, h], buf0.at[pl.ds(0, WPH)])
    accs = tuple(jnp.zeros((LANES,), jnp.float32) for _ in range(NVEC))
    for r in range(WPH):
        accs = tuple(
            accs[v] + buf0[r, pl.ds(v * LANES, LANES)] for v in range(NVEC)
        )
    for v in range(NVEC):
        obuf[0, pl.ds(v * LANES, LANES)] = accs[v] * inv
    pltpu.sync_copy(obuf.at[0], out_hbm.at[b_f, pl.ds(col0, HALF)])


@jax.jit
def _mean_pool(xs, lens):
    kern = pl.kernel(
        _body,
        out_type=(
            jax.ShapeDtypeStruct((B, D), jnp.float32),
            jax.ShapeDtypeStruct((B, 2, WPH, HALF), jnp.float32),
        ),
        mesh=plsc.VectorSubcoreMesh(core_axis_name="c", subcore_axis_name="s"),
        scratch_types=[
            pltpu.VMEM((LANES,), jnp.int32),
            pltpu.VMEM((CH, HALF), jnp.float32),
            pltpu.VMEM((CH, HALF), jnp.float32),
            pltpu.VMEM((SEQ_PER_SC, HALF), jnp.float32),
            pltpu.SemaphoreType.DMA,
            pltpu.SemaphoreType.DMA,
            pltpu.SemaphoreType.DMA,
        ],
    )
    out, _ = kern(xs, lens)
    return out


def kernel(xs, xs_len):
    return _mean_pool(xs, xs_len.astype(jnp.int32))


# speedup vs baseline: 1.1869x; 1.1869x over previous
import jax
import jax.numpy as jnp
from jax import lax
from jax.experimental import pallas as pl
from jax.experimental.pallas import tpu as pltpu
from jax.experimental.pallas import tpu_sc as plsc

B, L, D = 16, 2048, 1024
LANES = 16
HALF = D // 2
NVEC = HALF // LANES
CH = 64
BLK = 512
NSTEP = L // BLK
SEQ_PER_SC = B // 2


def _sc_body(xs_hbm, len_hbm, out_hbm, lbuf, buf0, buf1, obuf, sem0, sem1):
    c = lax.axis_index("c")
    s = lax.axis_index("s")
    b = c * SEQ_PER_SC + s // 2
    h = s % 2
    col0 = h * HALF

    pltpu.sync_copy(len_hbm, lbuf)
    lvec = lbuf[...]
    wk = s // 2
    l0 = lvec[0]
    l1 = lvec[SEQ_PER_SC]
    for j in range(1, SEQ_PER_SC):
        l0 = jnp.where(wk == j, lvec[j], l0)
        l1 = jnp.where(wk == j, lvec[SEQ_PER_SC + j], l1)
    l = jnp.where(c == 0, l0, l1)

    base = (l // BLK) * BLK
    n = l - base
    nch = (n + CH - 1) // CH

    def issue(i, buf, sem):
        pltpu.make_async_copy(
            xs_hbm.at[b, pl.ds(base + i * CH, CH), pl.ds(col0, HALF)],
            buf, sem,
        ).start()

    def wait(buf, sem):
        pltpu.make_async_copy(
            xs_hbm.at[b, pl.ds(0, CH), pl.ds(col0, HALF)], buf, sem
        ).wait()

    def accum(i, buf, accs):
        nv = jnp.clip(n - i * CH, 0, CH)

        def row_body(r, a):
            return tuple(
                a[v] + buf[r, pl.ds(v * LANES, LANES)] for v in range(NVEC)
            )

        return lax.fori_loop(0, nv, row_body, accs)

    @pl.when(0 < nch)
    def _():
        issue(0, buf0, sem0)

    @pl.when(1 < nch)
    def _():
        issue(1, buf1, sem1)

    def pair_body(i2, accs):
        ca = 2 * i2
        wait(buf0, sem0)
        accs = accum(ca, buf0, accs)

        @pl.when(ca + 2 < nch)
        def _():
            issue(ca + 2, buf0, sem0)

        @pl.when(ca + 1 < nch)
        def _():
            wait(buf1, sem1)

        accs = accum(ca + 1, buf1, accs)

        @pl.when(ca + 3 < nch)
        def _():
            issue(ca + 3, buf1, sem1)

        return accs

    accs = tuple(jnp.zeros((LANES,), jnp.float32) for _ in range(NVEC))
    accs = lax.fori_loop(0, (nch + 1) // 2, pair_body, accs)

    inv = 1.0 / jnp.full((LANES,), l).astype(jnp.float32)
    for v in range(NVEC):
        obuf[pl.ds(v * LANES, LANES)] = accs[v] * inv
    pltpu.sync_copy(obuf, out_hbm.at[b, pl.ds(col0, HALF)])


def _tc_body(len_ref, x_ref, o_ref):
    i = pl.program_id(1)
    l = len_ref[pl.program_id(0)]
    nfull = l // BLK

    @pl.when(i == 0)
    def _():
        o_ref[...] = jnp.zeros_like(o_ref)

    @pl.when(i < nfull)
    def _():
        inv = 1.0 / l.astype(jnp.float32)
        x = x_ref[0].reshape(BLK // 8, 8, D)
        o_ref[...] += (jnp.sum(x, axis=0) * inv)[None]


def _tc_index_map(b, i, len_ref):
    nfull = len_ref[b] // BLK
    return (b, jnp.minimum(i, jnp.maximum(nfull - 1, 0)), 0)


@jax.jit
def _mean_pool(xs, lens):
    sc_kern = pl.kernel(
        _sc_body,
        out_type=jax.ShapeDtypeStruct((B, D), jnp.float32),
        mesh=plsc.VectorSubcoreMesh(core_axis_name="c", subcore_axis_name="s"),
        scratch_types=[
            pltpu.VMEM((LANES,), jnp.int32),
            pltpu.VMEM((CH, HALF), jnp.float32),
            pltpu.VMEM((CH, HALF), jnp.float32),
            pltpu.VMEM((HALF,), jnp.float32),
            pltpu.SemaphoreType.DMA,
            pltpu.SemaphoreType.DMA,
        ],
    )
    sc_out = sc_kern(xs, lens)

    tc_out = pl.pallas_call(
        _tc_body,
        grid_spec=pltpu.PrefetchScalarGridSpec(
            num_scalar_prefetch=1,
            grid=(B, NSTEP),
            in_specs=[pl.BlockSpec((1, BLK, D), _tc_index_map)],
            out_specs=pl.BlockSpec((1, 8, D), lambda b, i, len_ref: (b, 0, 0)),
        ),
        out_shape=jax.ShapeDtypeStruct((B, 8, D), jnp.float32),
    )(lens, xs)

    return sc_out + jnp.sum(tc_out, axis=1)


def kernel(xs, xs_len):
    return _mean_pool(xs, xs_len.astype(jnp.int32))
